# Initial kernel scaffold; baseline (speedup 1.0000x reference)
import jax
import jax.numpy as jnp
from jax.experimental import pallas as pl


def kernel(positions, types, object_positions, object_colors, object_shapes, object_materials, object_sizes, question, question_table, pos_W, pos_b, color_table, shape_table, material_table, size_table, type_table):
    raise NotImplementedError("write your pallas kernel here")


# speedup vs baseline: 5.6983x; 5.6983x over previous
import functools

import jax
import jax.numpy as jnp
from jax import lax
from jax.experimental import pallas as pl
from jax.experimental.pallas import tpu as pltpu
from jax.experimental.pallas import tpu_sc as plsc

B = 16384
NOBJ = 10
Q = 50
D = 128

_NC = 2
_NS = 16
_NW = _NC * _NS

_CH = 128


def _sc_questions(q_ids, t_ids, qtab, ttab):
    n_tok = q_ids.shape[0]
    per_w = n_tok // _NW
    n_chunks = per_w // _CH
    mesh = plsc.VectorSubcoreMesh(
        core_axis_name="c", subcore_axis_name="s",
        num_cores=_NC, num_subcores=_NS)

    @functools.partial(
        pl.kernel,
        out_type=jax.ShapeDtypeStruct((n_tok, D), jnp.float32),
        mesh=mesh,
        scratch_types=[
            pltpu.VMEM((_CH,), jnp.int32),
            pltpu.VMEM((_CH, D), jnp.float32),
            pltpu.VMEM((3, D), jnp.float32),
            pltpu.VMEM((_CH,), jnp.int32),
            pltpu.SemaphoreType.DMA,
        ],
    )
    def k(qi_hbm, ti_hbm, qtab_hbm, ttab_hbm, out_hbm,
          ids_v, rows_v, ttab_v, t_v, sem):
        wid = lax.axis_index("s") * _NC + lax.axis_index("c")
        base = wid * per_w
        pltpu.sync_copy(ttab_hbm, ttab_v)
        t1_rows = [ttab_v[1, pl.ds(cb * 16, 16)] for cb in range(8)]
        t2_rows = [ttab_v[2, pl.ds(cb * 16, 16)] for cb in range(8)]

        def chunk(i, carry):
            off = base + i * _CH
            pltpu.sync_copy(qi_hbm.at[pl.ds(off, _CH)], ids_v)
            pltpu.sync_copy(ti_hbm.at[pl.ds(off, _CH)], t_v)
            pltpu.async_copy(qtab_hbm.at[ids_v], rows_v, sem).wait()

            def grp(g, c2):
                t16 = t_v[pl.ds(g * 16, 16)]
                for k in range(16):
                    j = g * 16 + k
                    w1 = (t16[k] == 1).astype(jnp.float32)
                    w2 = (t16[k] == 2).astype(jnp.float32)
                    for cb in range(8):
                        q = rows_v[j, pl.ds(cb * 16, 16)]
                        rows_v[j, pl.ds(cb * 16, 16)] = (
                            q + w1 * t1_rows[cb] + w2 * t2_rows[cb])
                return c2

            lax.fori_loop(0, _CH // 16, grp, 0)
            pltpu.sync_copy(rows_v, out_hbm.at[pl.ds(off, _CH)])
            return carry

        lax.fori_loop(0, n_chunks, chunk, 0)

    return k(q_ids, t_ids, qtab, ttab)


_BB = 256


def _tc_body(types_ref, pos_ref, c_ref, s_ref, m_ref, z_ref,
             ctab_ref, stab_ref, mtab_ref, ztab_ref, ttab_ref,
             wt_ref, b_ref,
             op_out, oc_out, os_out, om_out, oz_out, mask_out):
    shp = (_BB, NOBJ, D)

    def brow(tab_ref, v):
        return lax.broadcast_in_dim(tab_ref[v, :], shp, (2,))

    def bids(ids):
        return lax.broadcast_in_dim(ids, shp, (0, 1))

    tt = types_ref[:, :NOBJ]
    tt3 = bids(tt)
    otype = (jnp.where(tt3 == 1, brow(ttab_ref, 1), 0.0)
             + jnp.where(tt3 == 2, brow(ttab_ref, 2), 0.0))

    def lut(ids, tab_ref, nrows):
        ids3 = bids(ids)
        acc = otype
        for v in range(1, nrows):
            acc = acc + jnp.where(ids3 == v, brow(tab_ref, v), 0.0)
        return acc

    oc_out[...] = lut(c_ref[...], ctab_ref, 9)
    os_out[...] = lut(s_ref[...], stab_ref, 4)
    om_out[...] = lut(m_ref[...], mtab_ref, 3)
    oz_out[...] = lut(z_ref[...], ztab_ref, 3)

    def bpos(x):
        return lax.broadcast_in_dim(x, shp, (0, 1, 2))

    pos = pos_ref[...]
    op_out[...] = (bpos(pos[:, :, 0:1]) * brow(wt_ref, 0)
                   + bpos(pos[:, :, 1:2]) * brow(wt_ref, 1)
                   + bpos(pos[:, :, 2:3]) * brow(wt_ref, 2)
                   + brow(b_ref, 0) + otype)

    m = (tt == 1).astype(jnp.float32)
    qm = (types_ref[:, NOBJ:] == 2).astype(jnp.float32)
    mask_out[...] = jnp.concatenate([m, m, m, m, m, qm], axis=1)


def _tc_objects(types, obj_pos, colors, shapes, materials, sizes,
                ctab, stab, mtab, ztab, ttab, pos_wt, pos_b2):
    grid = B // _BB
    blk2 = lambda w: pl.BlockSpec((_BB, w), lambda i: (i, 0))
    tabspec = lambda t: pl.BlockSpec(t.shape, lambda i: (0,) * t.ndim)
    out3 = jax.ShapeDtypeStruct((B, NOBJ, D), jnp.float32)
    return pl.pallas_call(
        _tc_body,
        grid=(grid,),
        in_specs=[
            blk2(NOBJ + Q),
            pl.BlockSpec((_BB, NOBJ, 3), lambda i: (i, 0, 0)),
            blk2(NOBJ), blk2(NOBJ), blk2(NOBJ), blk2(NOBJ),
            tabspec(ctab), tabspec(stab), tabspec(mtab), tabspec(ztab),
            tabspec(ttab), tabspec(pos_wt), tabspec(pos_b2),
        ],
        out_specs=[
            pl.BlockSpec((_BB, NOBJ, D), lambda i: (i, 0, 0)),
            pl.BlockSpec((_BB, NOBJ, D), lambda i: (i, 0, 0)),
            pl.BlockSpec((_BB, NOBJ, D), lambda i: (i, 0, 0)),
            pl.BlockSpec((_BB, NOBJ, D), lambda i: (i, 0, 0)),
            pl.BlockSpec((_BB, NOBJ, D), lambda i: (i, 0, 0)),
            blk2(5 * NOBJ + Q),
        ],
        out_shape=[out3, out3, out3, out3, out3,
                   jax.ShapeDtypeStruct((B, 5 * NOBJ + Q), jnp.float32)],
    )(types, obj_pos, colors, shapes, materials, sizes,
      ctab, stab, mtab, ztab, ttab, pos_wt, pos_b2)


def kernel(positions, types, object_positions, object_colors, object_shapes,
           object_materials, object_sizes, question, question_table, pos_W,
           pos_b, color_table, shape_table, material_table, size_table,
           type_table):
    types = types.astype(jnp.int32)
    q_ids = question.astype(jnp.int32).reshape(-1)
    t_ids = types[:, NOBJ:].reshape(-1)

    questions = _sc_questions(q_ids, t_ids, question_table,
                              type_table).reshape(B, Q, D)

    op, oc, osh, om, oz, mask = _tc_objects(
        types, object_positions,
        object_colors.astype(jnp.int32), object_shapes.astype(jnp.int32),
        object_materials.astype(jnp.int32), object_sizes.astype(jnp.int32),
        color_table, shape_table, material_table, size_table, type_table,
        pos_W.T, pos_b.reshape(1, D))

    mixed_mask = mask.reshape(B, 1, 1, 5 * NOBJ + Q)
    return (op, oc, osh, om, oz, questions, mixed_mask)


# speedup vs baseline: 5.7077x; 1.0016x over previous
import functools

import jax
import jax.numpy as jnp
from jax import lax
from jax.experimental import pallas as pl
from jax.experimental.pallas import tpu as pltpu
from jax.experimental.pallas import tpu_sc as plsc

B = 16384
NOBJ = 10
Q = 50
D = 128

_NC = 2
_NS = 16
_NW = _NC * _NS

_CH = 128


def _sc_questions(q_ids, t_ids, qtab, ttab):
    n_tok = q_ids.shape[0]
    per_w = n_tok // _NW
    n_chunks = per_w // _CH
    mesh = plsc.VectorSubcoreMesh(
        core_axis_name="c", subcore_axis_name="s",
        num_cores=_NC, num_subcores=_NS)

    @functools.partial(
        pl.kernel,
        out_type=jax.ShapeDtypeStruct((n_tok, D), jnp.float32),
        mesh=mesh,
        scratch_types=[
            pltpu.VMEM((_CH,), jnp.int32),
            pltpu.VMEM((_CH, D), jnp.float32),
            pltpu.VMEM((3, D), jnp.float32),
            pltpu.VMEM((_CH,), jnp.int32),
            pltpu.SemaphoreType.DMA,
        ],
        compiler_params=pltpu.CompilerParams(use_tc_tiling_on_sc=True),
    )
    def k(qi_hbm, ti_hbm, qtab_hbm, ttab_hbm, out_hbm,
          ids_v, rows_v, ttab_v, t_v, sem):
        wid = lax.axis_index("s") * _NC + lax.axis_index("c")
        base = wid * per_w
        pltpu.sync_copy(ttab_hbm, ttab_v)
        t1_rows = [ttab_v[1, pl.ds(cb * 16, 16)] for cb in range(8)]
        t2_rows = [ttab_v[2, pl.ds(cb * 16, 16)] for cb in range(8)]

        def chunk(i, carry):
            off = base + i * _CH
            pltpu.sync_copy(qi_hbm.at[pl.ds(off, _CH)], ids_v)
            pltpu.sync_copy(ti_hbm.at[pl.ds(off, _CH)], t_v)
            pltpu.async_copy(qtab_hbm.at[ids_v], rows_v, sem).wait()

            def grp(g, c2):
                t16 = t_v[pl.ds(g * 16, 16)]
                for k in range(16):
                    j = g * 16 + k
                    w1 = (t16[k] == 1).astype(jnp.float32)
                    w2 = (t16[k] == 2).astype(jnp.float32)
                    for cb in range(8):
                        q = rows_v[j, pl.ds(cb * 16, 16)]
                        rows_v[j, pl.ds(cb * 16, 16)] = (
                            q + w1 * t1_rows[cb] + w2 * t2_rows[cb])
                return c2

            lax.fori_loop(0, _CH // 16, grp, 0)
            pltpu.sync_copy(rows_v, out_hbm.at[pl.ds(off, _CH)])
            return carry

        lax.fori_loop(0, n_chunks, chunk, 0)

    return k(q_ids, t_ids, qtab, ttab)


_BB = 256


def _tc_body(types_ref, pos_ref, c_ref, s_ref, m_ref, z_ref,
             ctab_ref, stab_ref, mtab_ref, ztab_ref, ttab_ref,
             wt_ref, b_ref,
             op_out, oc_out, os_out, om_out, oz_out, mask_out):
    shp = (_BB, NOBJ, D)

    def brow(tab_ref, v):
        return lax.broadcast_in_dim(tab_ref[v, :], shp, (2,))

    def bids(ids):
        return lax.broadcast_in_dim(ids, shp, (0, 1))

    tt = types_ref[:, :NOBJ]
    tt3 = bids(tt)
    otype = (jnp.where(tt3 == 1, brow(ttab_ref, 1), 0.0)
             + jnp.where(tt3 == 2, brow(ttab_ref, 2), 0.0))

    def lut(ids, tab_ref, nrows):
        ids3 = bids(ids)
        acc = otype
        for v in range(1, nrows):
            acc = acc + jnp.where(ids3 == v, brow(tab_ref, v), 0.0)
        return acc

    oc_out[...] = lut(c_ref[...], ctab_ref, 9)
    os_out[...] = lut(s_ref[...], stab_ref, 4)
    om_out[...] = lut(m_ref[...], mtab_ref, 3)
    oz_out[...] = lut(z_ref[...], ztab_ref, 3)

    def bpos(x):
        return lax.broadcast_in_dim(x, shp, (0, 1, 2))

    pos = pos_ref[...]
    op_out[...] = (bpos(pos[:, :, 0:1]) * brow(wt_ref, 0)
                   + bpos(pos[:, :, 1:2]) * brow(wt_ref, 1)
                   + bpos(pos[:, :, 2:3]) * brow(wt_ref, 2)
                   + brow(b_ref, 0) + otype)

    m = (tt == 1).astype(jnp.float32)
    qm = (types_ref[:, NOBJ:] == 2).astype(jnp.float32)
    mask_out[...] = jnp.concatenate([m, m, m, m, m, qm], axis=1)


def _tc_objects(types, obj_pos, colors, shapes, materials, sizes,
                ctab, stab, mtab, ztab, ttab, pos_wt, pos_b2):
    grid = B // _BB
    blk2 = lambda w: pl.BlockSpec((_BB, w), lambda i: (i, 0))
    tabspec = lambda t: pl.BlockSpec(t.shape, lambda i: (0,) * t.ndim)
    out3 = jax.ShapeDtypeStruct((B, NOBJ, D), jnp.float32)
    return pl.pallas_call(
        _tc_body,
        grid=(grid,),
        in_specs=[
            blk2(NOBJ + Q),
            pl.BlockSpec((_BB, NOBJ, 3), lambda i: (i, 0, 0)),
            blk2(NOBJ), blk2(NOBJ), blk2(NOBJ), blk2(NOBJ),
            tabspec(ctab), tabspec(stab), tabspec(mtab), tabspec(ztab),
            tabspec(ttab), tabspec(pos_wt), tabspec(pos_b2),
        ],
        out_specs=[
            pl.BlockSpec((_BB, NOBJ, D), lambda i: (i, 0, 0)),
            pl.BlockSpec((_BB, NOBJ, D), lambda i: (i, 0, 0)),
            pl.BlockSpec((_BB, NOBJ, D), lambda i: (i, 0, 0)),
            pl.BlockSpec((_BB, NOBJ, D), lambda i: (i, 0, 0)),
            pl.BlockSpec((_BB, NOBJ, D), lambda i: (i, 0, 0)),
            blk2(5 * NOBJ + Q),
        ],
        out_shape=[out3, out3, out3, out3, out3,
                   jax.ShapeDtypeStruct((B, 5 * NOBJ + Q), jnp.float32)],
    )(types, obj_pos, colors, shapes, materials, sizes,
      ctab, stab, mtab, ztab, ttab, pos_wt, pos_b2)


def kernel(positions, types, object_positions, object_colors, object_shapes,
           object_materials, object_sizes, question, question_table, pos_W,
           pos_b, color_table, shape_table, material_table, size_table,
           type_table):
    types = types.astype(jnp.int32)
    q_ids = question.astype(jnp.int32).reshape(-1)
    t_ids = types[:, NOBJ:].reshape(-1)

    questions = _sc_questions(q_ids, t_ids, question_table,
                              type_table).reshape(B, Q, D)

    op, oc, osh, om, oz, mask = _tc_objects(
        types, object_positions,
        object_colors.astype(jnp.int32), object_shapes.astype(jnp.int32),
        object_materials.astype(jnp.int32), object_sizes.astype(jnp.int32),
        color_table, shape_table, material_table, size_table, type_table,
        pos_W.T, pos_b.reshape(1, D))

    mixed_mask = mask.reshape(B, 1, 1, 5 * NOBJ + Q)
    return (op, oc, osh, om, oz, questions, mixed_mask)


# speedup vs baseline: 6.4534x; 1.1306x over previous
import functools

import jax
import jax.numpy as jnp
from jax import lax
from jax.experimental import pallas as pl
from jax.experimental.pallas import tpu as pltpu
from jax.experimental.pallas import tpu_sc as plsc

B = 16384
NOBJ = 10
Q = 50
D = 128

_NC = 2
_NS = 16
_NW = _NC * _NS

_CH = 128


def _sc_questions(q_ids, t_ids, qtab, ttab):
    n_tok = q_ids.shape[0]
    per_w = n_tok // _NW
    n_chunks = per_w // _CH
    mesh = plsc.VectorSubcoreMesh(
        core_axis_name="c", subcore_axis_name="s",
        num_cores=_NC, num_subcores=_NS)

    @functools.partial(
        pl.kernel,
        out_type=jax.ShapeDtypeStruct((n_tok, D), jnp.float32),
        mesh=mesh,
        scratch_types=[
            pltpu.VMEM((per_w,), jnp.int32),
            pltpu.VMEM((per_w,), jnp.int32),
            pltpu.VMEM((2, _CH, D), jnp.float32),
            pltpu.VMEM((2, _CH, D), jnp.float32),
            pltpu.VMEM((3, D), jnp.float32),
            pltpu.SemaphoreType.DMA,
            pltpu.SemaphoreType.DMA,
            pltpu.SemaphoreType.DMA,
            pltpu.SemaphoreType.DMA,
        ],
        compiler_params=pltpu.CompilerParams(use_tc_tiling_on_sc=True),
    )
    def k(qi_hbm, ti_hbm, qtab_hbm, ttab_hbm, out_hbm,
          ids_all, t_all, rows2, st2, ttab_v, g0, g1, s0, s1):
        wid = lax.axis_index("s") * _NC + lax.axis_index("c")
        base = wid * per_w
        pltpu.sync_copy(qi_hbm.at[pl.ds(base, per_w)], ids_all)
        pltpu.sync_copy(ti_hbm.at[pl.ds(base, per_w)], t_all)
        pltpu.sync_copy(ttab_hbm, ttab_v)
        t1_rows = [ttab_v[1, pl.ds(cb * 16, 16)] for cb in range(8)]
        t2_rows = [ttab_v[2, pl.ds(cb * 16, 16)] for cb in range(8)]
        gs = (g0, g1)
        ss = (s0, s1)

        def gather(i, b, sem):
            return pltpu.async_copy(
                qtab_hbm.at[ids_all.at[pl.ds(i * _CH, _CH)]],
                rows2.at[b], sem)

        def store(i, b, sem):
            return pltpu.async_copy(
                st2.at[b], out_hbm.at[pl.ds(base + i * _CH, _CH)], sem)

        gather(0, 0, g0)
        gather(1, 1, g1)

        def body(kk, carry):
            for b in range(2):
                i = 2 * kk + b
                pltpu.make_async_copy(
                    qtab_hbm.at[ids_all.at[pl.ds(i * _CH, _CH)]],
                    rows2.at[b], gs[b]).wait()

                @pl.when(kk > 0)
                def _():
                    pltpu.make_async_copy(
                        st2.at[b],
                        out_hbm.at[pl.ds(base + (i - 2) * _CH, _CH)],
                        ss[b]).wait()

                def grp(g, c2):
                    t16 = t_all[pl.ds(i * _CH + g * 16, 16)]
                    for kj in range(16):
                        j = g * 16 + kj
                        w1 = (t16[kj] == 1).astype(jnp.float32)
                        w2 = (t16[kj] == 2).astype(jnp.float32)
                        for cb in range(8):
                            q = rows2[b, j, pl.ds(cb * 16, 16)]
                            st2[b, j, pl.ds(cb * 16, 16)] = (
                                q + w1 * t1_rows[cb] + w2 * t2_rows[cb])
                    return c2

                lax.fori_loop(0, _CH // 16, grp, 0)
                store(i, b, ss[b])

                @pl.when(i + 2 < n_chunks)
                def _():
                    gather(i + 2, b, gs[b])
            return carry

        lax.fori_loop(0, n_chunks // 2, body, 0)
        for b in range(2):
            pltpu.make_async_copy(
                st2.at[b],
                out_hbm.at[pl.ds(base + (n_chunks - 2 + b) * _CH, _CH)],
                ss[b]).wait()

    return k(q_ids, t_ids, qtab, ttab)


_BB = 256


def _tc_body(types_ref, pos_ref, c_ref, s_ref, m_ref, z_ref,
             ctab_ref, stab_ref, mtab_ref, ztab_ref, ttab_ref,
             wt_ref, b_ref,
             op_out, oc_out, os_out, om_out, oz_out, mask_out):
    shp = (_BB, NOBJ, D)

    def brow(tab_ref, v):
        return lax.broadcast_in_dim(tab_ref[v, :], shp, (2,))

    def bids(ids):
        return lax.broadcast_in_dim(ids, shp, (0, 1))

    tt = types_ref[:, :NOBJ]
    tt3 = bids(tt)
    otype = (jnp.where(tt3 == 1, brow(ttab_ref, 1), 0.0)
             + jnp.where(tt3 == 2, brow(ttab_ref, 2), 0.0))

    def lut(ids, tab_ref, nrows):
        ids3 = bids(ids)
        acc = otype
        for v in range(1, nrows):
            acc = acc + jnp.where(ids3 == v, brow(tab_ref, v), 0.0)
        return acc

    oc_out[...] = lut(c_ref[...], ctab_ref, 9)
    os_out[...] = lut(s_ref[...], stab_ref, 4)
    om_out[...] = lut(m_ref[...], mtab_ref, 3)
    oz_out[...] = lut(z_ref[...], ztab_ref, 3)

    def bpos(x):
        return lax.broadcast_in_dim(x, shp, (0, 1, 2))

    pos = pos_ref[...]
    op_out[...] = (bpos(pos[:, :, 0:1]) * brow(wt_ref, 0)
                   + bpos(pos[:, :, 1:2]) * brow(wt_ref, 1)
                   + bpos(pos[:, :, 2:3]) * brow(wt_ref, 2)
                   + brow(b_ref, 0) + otype)

    m = (tt == 1).astype(jnp.float32)
    qm = (types_ref[:, NOBJ:] == 2).astype(jnp.float32)
    mask_out[...] = jnp.concatenate([m, m, m, m, m, qm], axis=1)


def _tc_objects(types, obj_pos, colors, shapes, materials, sizes,
                ctab, stab, mtab, ztab, ttab, pos_wt, pos_b2):
    grid = B // _BB
    blk2 = lambda w: pl.BlockSpec((_BB, w), lambda i: (i, 0))
    tabspec = lambda t: pl.BlockSpec(t.shape, lambda i: (0,) * t.ndim)
    out3 = jax.ShapeDtypeStruct((B, NOBJ, D), jnp.float32)
    return pl.pallas_call(
        _tc_body,
        grid=(grid,),
        in_specs=[
            blk2(NOBJ + Q),
            pl.BlockSpec((_BB, NOBJ, 3), lambda i: (i, 0, 0)),
            blk2(NOBJ), blk2(NOBJ), blk2(NOBJ), blk2(NOBJ),
            tabspec(ctab), tabspec(stab), tabspec(mtab), tabspec(ztab),
            tabspec(ttab), tabspec(pos_wt), tabspec(pos_b2),
        ],
        out_specs=[
            pl.BlockSpec((_BB, NOBJ, D), lambda i: (i, 0, 0)),
            pl.BlockSpec((_BB, NOBJ, D), lambda i: (i, 0, 0)),
            pl.BlockSpec((_BB, NOBJ, D), lambda i: (i, 0, 0)),
            pl.BlockSpec((_BB, NOBJ, D), lambda i: (i, 0, 0)),
            pl.BlockSpec((_BB, NOBJ, D), lambda i: (i, 0, 0)),
            blk2(5 * NOBJ + Q),
        ],
        out_shape=[out3, out3, out3, out3, out3,
                   jax.ShapeDtypeStruct((B, 5 * NOBJ + Q), jnp.float32)],
    )(types, obj_pos, colors, shapes, materials, sizes,
      ctab, stab, mtab, ztab, ttab, pos_wt, pos_b2)


def kernel(positions, types, object_positions, object_colors, object_shapes,
           object_materials, object_sizes, question, question_table, pos_W,
           pos_b, color_table, shape_table, material_table, size_table,
           type_table):
    types = types.astype(jnp.int32)
    q_ids = question.astype(jnp.int32).reshape(-1)
    t_ids = types[:, NOBJ:].reshape(-1)

    questions = _sc_questions(q_ids, t_ids, question_table,
                              type_table).reshape(B, Q, D)

    op, oc, osh, om, oz, mask = _tc_objects(
        types, object_positions,
        object_colors.astype(jnp.int32), object_shapes.astype(jnp.int32),
        object_materials.astype(jnp.int32), object_sizes.astype(jnp.int32),
        color_table, shape_table, material_table, size_table, type_table,
        pos_W.T, pos_b.reshape(1, D))

    mixed_mask = mask.reshape(B, 1, 1, 5 * NOBJ + Q)
    return (op, oc, osh, om, oz, questions, mixed_mask)
